# Initial kernel scaffold; baseline (speedup 1.0000x reference)
import jax
import jax.numpy as jnp
from jax.experimental import pallas as pl


def kernel(feat0, feat1, feat2, feat3, feat4, im_info, w_conv, b_conv, w_cls, b_cls, w_box, b_box):
    raise NotImplementedError("write your pallas kernel here")


# speedup vs baseline: 1.0352x; 1.0352x over previous
import functools

import numpy as np
import jax
import jax.numpy as jnp
from jax import lax
from jax.experimental import pallas as pl
from jax.experimental.pallas import tpu as pltpu

_RATIOS = (0.5, 1.0, 2.0)
_A = 3
_PRE_NMS = 1000
_POST_NMS = 1000
_NMS_THRESH = 0.7
_BBOX_CLIP = float(np.log(1000.0 / 16.0))


def _anchor_consts(stride):
    size = np.float32(stride) * np.float32(4.0)
    area = np.float32(size * size)
    w2 = np.zeros((8,), np.float32)
    h2 = np.zeros((8,), np.float32)
    for a, r in enumerate(_RATIOS):
        ws = np.float32(np.sqrt(np.float32(area / np.float32(r))))
        hs = np.float32(ws * np.float32(r))
        w2[a] = np.float32(ws / np.float32(2.0))
        h2[a] = np.float32(hs / np.float32(2.0))
    return w2, h2


def _head_body(xprev_ref, xcur_ref, xnext_ref, w9_ref, bconv_ref, whead_ref,
               bhead_ref, aconst_ref, iminfo_ref, probs_ref, boxes_ref,
               *, H, W, SP, stride):
    b = pl.program_id(0)
    s = pl.program_id(1)

    xwin = jnp.concatenate(
        [xprev_ref[0], xcur_ref[0], xnext_ref[0]], axis=0)

    acc = jnp.zeros((SP, 256), jnp.float32) + bconv_ref[0][None, :]
    pcol = jax.lax.broadcasted_iota(jnp.int32, (SP, 1), 0)
    wcol = pcol % W
    for dy in (-1, 0, 1):
        for dx in (-1, 0, 1):
            o = dy * W + dx
            t = (dy + 1) * 3 + (dx + 1)
            xs = xwin[SP + o:2 * SP + o, :]
            pp = jnp.dot(xs, w9_ref[t],
                         preferred_element_type=jnp.float32)
            if dx == -1:
                m = (wcol > 0).astype(jnp.float32)
                pp = pp * m
            elif dx == 1:
                m = (wcol < (W - 1)).astype(jnp.float32)
                pp = pp * m
            acc = acc + pp
    tact = jnp.maximum(acc, 0.0)
    head = jnp.dot(tact, whead_ref[...],
                   preferred_element_type=jnp.float32) + bhead_ref[0][None, :]

    l0 = head[:, 0:8]
    l1 = head[:, 8:16]
    dxv = head[:, 16:24]
    dyv = head[:, 24:32]
    dwv = head[:, 32:40]
    dhv = head[:, 40:48]

    m = jnp.maximum(l0, l1)
    e0 = jnp.exp(l0 - m)
    e1 = jnp.exp(l1 - m)
    probs_ref[0] = e1 / (e0 + e1)

    nw2 = aconst_ref[0:1, :]
    nh2 = aconst_ref[1:2, :]
    pw2c = aconst_ref[2:3, :]
    ph2c = aconst_ref[3:4, :]

    p_global = s * SP + jax.lax.broadcasted_iota(jnp.int32, (SP, 1), 0)
    wq = (p_global % W).astype(jnp.float32)
    hq = (p_global // W).astype(jnp.float32)
    cx = (wq + 0.5) * np.float32(stride)
    cy = (hq + 0.5) * np.float32(stride)
    x1a = cx + nw2
    y1a = cy + nh2
    x2a = cx + pw2c
    y2a = cy + ph2c
    wa = x2a - x1a
    ha = y2a - y1a
    xa = x1a + wa * 0.5
    ya = y1a + ha * 0.5

    dwc = jnp.minimum(dwv, np.float32(_BBOX_CLIP))
    dhc = jnp.minimum(dhv, np.float32(_BBOX_CLIP))
    px = dxv * wa + xa
    py = dyv * ha + ya
    pw = jnp.exp(dwc) * wa
    ph = jnp.exp(dhc) * ha
    x1 = px - pw * 0.5
    y1 = py - ph * 0.5
    x2 = px + pw * 0.5
    y2 = py + ph * 0.5

    h_im = iminfo_ref[b, 0]
    w_im = iminfo_ref[b, 1]
    x1 = jnp.clip(x1, 0.0, w_im)
    y1 = jnp.clip(y1, 0.0, h_im)
    x2 = jnp.clip(x2, 0.0, w_im)
    y2 = jnp.clip(y2, 0.0, h_im)

    boxes_ref[0, :, 0:4] = x1[:, 0:4]
    boxes_ref[0, :, 4:8] = y1[:, 0:4]
    boxes_ref[0, :, 8:12] = x2[:, 0:4]
    boxes_ref[0, :, 12:16] = y2[:, 0:4]


def _head_level(x, w9, bconv, whead, bhead, im_info, H, W, stride):
    P = H * W
    SP = min(P, 2048)
    nstrips = P // SP
    xr = jnp.transpose(x, (0, 2, 3, 1)).reshape(2, P, 256)
    xp = jnp.pad(xr, ((0, 0), (SP, SP), (0, 0)))
    w2, h2 = _anchor_consts(stride)
    aconst = jnp.asarray(np.stack([-w2, -h2, w2, h2]))

    blk = pl.BlockSpec((1, SP, 256), lambda b, s: (b, s, 0))
    blk_n = pl.BlockSpec((1, SP, 256), lambda b, s: (b, s + 1, 0))
    blk_nn = pl.BlockSpec((1, SP, 256), lambda b, s: (b, s + 2, 0))
    body = functools.partial(_head_body, H=H, W=W, SP=SP, stride=stride)
    probs, boxes = pl.pallas_call(
        body,
        grid=(2, nstrips),
        in_specs=[
            blk, blk_n, blk_nn,
            pl.BlockSpec((9, 256, 256), lambda b, s: (0, 0, 0)),
            pl.BlockSpec((1, 256), lambda b, s: (0, 0)),
            pl.BlockSpec((256, 48), lambda b, s: (0, 0)),
            pl.BlockSpec((1, 48), lambda b, s: (0, 0)),
            pl.BlockSpec((4, 8), lambda b, s: (0, 0)),
            pl.BlockSpec(memory_space=pltpu.SMEM),
        ],
        out_specs=[
            pl.BlockSpec((1, SP, 8), lambda b, s: (b, s, 0)),
            pl.BlockSpec((1, SP, 16), lambda b, s: (b, s, 0)),
        ],
        out_shape=[
            jax.ShapeDtypeStruct((2, P, 8), jnp.float32),
            jax.ShapeDtypeStruct((2, P, 16), jnp.float32),
        ],
    )(xp, xp, xp, w9, bconv, whead, bhead, aconst, im_info)
    return probs, boxes


def _pack_weights(w_conv, b_conv, w_cls, b_cls, w_box, b_box):
    w9 = jnp.transpose(w_conv, (2, 3, 1, 0)).reshape(9, 256, 256)
    bconv = b_conv.reshape(1, 256)
    whead = jnp.zeros((256, 48), jnp.float32)
    bhead = jnp.zeros((1, 48), jnp.float32)
    wc = w_cls[:, :, 0, 0]
    wb = w_box[:, :, 0, 0]
    for a in range(_A):
        whead = whead.at[:, 0 + a].set(wc[2 * a])
        whead = whead.at[:, 8 + a].set(wc[2 * a + 1])
        bhead = bhead.at[0, 0 + a].set(b_cls[2 * a])
        bhead = bhead.at[0, 8 + a].set(b_cls[2 * a + 1])
        for c in range(4):
            whead = whead.at[:, 16 + 8 * c + a].set(wb[4 * a + c])
            bhead = bhead.at[0, 16 + 8 * c + a].set(b_box[4 * a + c])
    return w9, bconv, whead, bhead


def _nms_tail(boxes_cat, scores_cat):
    order = jnp.argsort(-scores_cat)
    b = boxes_cat[order]
    x1, y1, x2, y2 = b[:, 0], b[:, 1], b[:, 2], b[:, 3]
    areas = jnp.maximum(x2 - x1, 0.0) * jnp.maximum(y2 - y1, 0.0)
    n = b.shape[0]
    idx = jnp.arange(n)

    def body(i, keep):
        xx1 = jnp.maximum(x1[i], x1)
        yy1 = jnp.maximum(y1[i], y1)
        xx2 = jnp.minimum(x2[i], x2)
        yy2 = jnp.minimum(y2[i], y2)
        inter = jnp.maximum(xx2 - xx1, 0.0) * jnp.maximum(yy2 - yy1, 0.0)
        iou = inter / jnp.maximum(areas[i] + areas - inter, 1e-9)
        suppress = (iou > _NMS_THRESH) & (idx > i) & keep[i]
        return keep & (~suppress)

    keep = lax.fori_loop(0, n, body, jnp.ones((n,), dtype=bool))
    sorted_scores = scores_cat[order]
    masked = jnp.where(keep, sorted_scores, -1.0)
    _, sel = lax.top_k(masked, _POST_NMS)
    return b[sel]


def kernel(feat0, feat1, feat2, feat3, feat4, im_info,
           w_conv, b_conv, w_cls, b_cls, w_box, b_box):
    feats = [feat0, feat1, feat2, feat3, feat4]
    w9, bconv, whead, bhead = _pack_weights(
        w_conv, b_conv, w_cls, b_cls, w_box, b_box)

    num_levels = len(feats)
    probs_l, boxes_l = [], []
    off = 2 ** (num_levels - 1)
    for x in feats:
        H, W = x.shape[2], x.shape[3]
        stride = 4 * off
        off //= 2
        p, bx = _head_level(x, w9, bconv, whead, bhead, im_info, H, W, stride)
        probs_l.append(p)
        boxes_l.append(bx)

    rois = []
    for bi in range(2):
        boxes_all, scores_all = [], []
        for l, x in enumerate(feats):
            H, W = x.shape[2], x.shape[3]
            P = H * W
            prob = probs_l[l][bi, :, 0:3].reshape(P * _A)
            k = min(_PRE_NMS, P * _A)
            top_s, top_i = lax.top_k(prob, k)
            p_i = top_i // _A
            a_i = top_i % _A
            bflat = boxes_l[l][bi].reshape(P * 16)
            base = p_i * 16 + a_i
            dec = jnp.stack([bflat[base], bflat[base + 4],
                             bflat[base + 8], bflat[base + 12]], axis=1)
            boxes_all.append(dec)
            scores_all.append(top_s)
        boxes_cat = jnp.concatenate(boxes_all, axis=0)
        scores_cat = jnp.concatenate(scores_all, axis=0)
        final = _nms_tail(boxes_cat, scores_cat)
        bcol = jnp.full((_POST_NMS, 1), float(bi), dtype=final.dtype)
        rois.append(jnp.concatenate([bcol, final], axis=1))
    return jnp.concatenate(rois, axis=0)


# speedup vs baseline: 27.6939x; 26.7517x over previous
import functools

import numpy as np
import jax
import jax.numpy as jnp
from jax import lax
from jax.experimental import pallas as pl
from jax.experimental.pallas import tpu as pltpu

_RATIOS = (0.5, 1.0, 2.0)
_A = 3
_PRE_NMS = 1000
_POST_NMS = 1000
_NMS_THRESH = 0.7
_BBOX_CLIP = float(np.log(1000.0 / 16.0))


def _anchor_consts(stride):
    size = np.float32(stride) * np.float32(4.0)
    area = np.float32(size * size)
    w2 = np.zeros((8,), np.float32)
    h2 = np.zeros((8,), np.float32)
    for a, r in enumerate(_RATIOS):
        ws = np.float32(np.sqrt(np.float32(area / np.float32(r))))
        hs = np.float32(ws * np.float32(r))
        w2[a] = np.float32(ws / np.float32(2.0))
        h2[a] = np.float32(hs / np.float32(2.0))
    return w2, h2


def _head_body(xprev_ref, xcur_ref, xnext_ref, w9_ref, bconv_ref, whead_ref,
               bhead_ref, aconst_ref, iminfo_ref, probs_ref, boxes_ref,
               *, H, W, SP, stride):
    b = pl.program_id(0)
    s = pl.program_id(1)

    xwin = jnp.concatenate(
        [xprev_ref[0], xcur_ref[0], xnext_ref[0]], axis=0)

    acc = jnp.zeros((SP, 256), jnp.float32) + bconv_ref[0][None, :]
    pcol = jax.lax.broadcasted_iota(jnp.int32, (SP, 1), 0)
    wcol = pcol % W
    for dy in (-1, 0, 1):
        for dx in (-1, 0, 1):
            o = dy * W + dx
            t = (dy + 1) * 3 + (dx + 1)
            xs = xwin[SP + o:2 * SP + o, :]
            pp = jnp.dot(xs, w9_ref[t],
                         preferred_element_type=jnp.float32)
            if dx == -1:
                m = (wcol > 0).astype(jnp.float32)
                pp = pp * m
            elif dx == 1:
                m = (wcol < (W - 1)).astype(jnp.float32)
                pp = pp * m
            acc = acc + pp
    tact = jnp.maximum(acc, 0.0)
    head = jnp.dot(tact, whead_ref[...],
                   preferred_element_type=jnp.float32) + bhead_ref[0][None, :]

    l0 = head[:, 0:8]
    l1 = head[:, 8:16]
    dxv = head[:, 16:24]
    dyv = head[:, 24:32]
    dwv = head[:, 32:40]
    dhv = head[:, 40:48]

    m = jnp.maximum(l0, l1)
    e0 = jnp.exp(l0 - m)
    e1 = jnp.exp(l1 - m)
    probs_ref[0] = e1 / (e0 + e1)

    nw2 = aconst_ref[0:1, :]
    nh2 = aconst_ref[1:2, :]
    pw2c = aconst_ref[2:3, :]
    ph2c = aconst_ref[3:4, :]

    p_global = s * SP + jax.lax.broadcasted_iota(jnp.int32, (SP, 1), 0)
    wq = (p_global % W).astype(jnp.float32)
    hq = (p_global // W).astype(jnp.float32)
    cx = (wq + 0.5) * np.float32(stride)
    cy = (hq + 0.5) * np.float32(stride)
    x1a = cx + nw2
    y1a = cy + nh2
    x2a = cx + pw2c
    y2a = cy + ph2c
    wa = x2a - x1a
    ha = y2a - y1a
    xa = x1a + wa * 0.5
    ya = y1a + ha * 0.5

    dwc = jnp.minimum(dwv, np.float32(_BBOX_CLIP))
    dhc = jnp.minimum(dhv, np.float32(_BBOX_CLIP))
    px = dxv * wa + xa
    py = dyv * ha + ya
    pw = jnp.exp(dwc) * wa
    ph = jnp.exp(dhc) * ha
    x1 = px - pw * 0.5
    y1 = py - ph * 0.5
    x2 = px + pw * 0.5
    y2 = py + ph * 0.5

    h_im = iminfo_ref[b, 0]
    w_im = iminfo_ref[b, 1]
    x1 = jnp.clip(x1, 0.0, w_im)
    y1 = jnp.clip(y1, 0.0, h_im)
    x2 = jnp.clip(x2, 0.0, w_im)
    y2 = jnp.clip(y2, 0.0, h_im)

    boxes_ref[0, :, 0:4] = x1[:, 0:4]
    boxes_ref[0, :, 4:8] = y1[:, 0:4]
    boxes_ref[0, :, 8:12] = x2[:, 0:4]
    boxes_ref[0, :, 12:16] = y2[:, 0:4]


def _head_level(x, w9, bconv, whead, bhead, im_info, H, W, stride):
    P = H * W
    SP = min(P, 2048)
    nstrips = P // SP
    xr = jnp.transpose(x, (0, 2, 3, 1)).reshape(2, P, 256)
    xp = jnp.pad(xr, ((0, 0), (SP, SP), (0, 0)))
    w2, h2 = _anchor_consts(stride)
    aconst = jnp.asarray(np.stack([-w2, -h2, w2, h2]))

    blk = pl.BlockSpec((1, SP, 256), lambda b, s: (b, s, 0))
    blk_n = pl.BlockSpec((1, SP, 256), lambda b, s: (b, s + 1, 0))
    blk_nn = pl.BlockSpec((1, SP, 256), lambda b, s: (b, s + 2, 0))
    body = functools.partial(_head_body, H=H, W=W, SP=SP, stride=stride)
    probs, boxes = pl.pallas_call(
        body,
        grid=(2, nstrips),
        in_specs=[
            blk, blk_n, blk_nn,
            pl.BlockSpec((9, 256, 256), lambda b, s: (0, 0, 0)),
            pl.BlockSpec((1, 256), lambda b, s: (0, 0)),
            pl.BlockSpec((256, 48), lambda b, s: (0, 0)),
            pl.BlockSpec((1, 48), lambda b, s: (0, 0)),
            pl.BlockSpec((4, 8), lambda b, s: (0, 0)),
            pl.BlockSpec(memory_space=pltpu.SMEM),
        ],
        out_specs=[
            pl.BlockSpec((1, SP, 8), lambda b, s: (b, s, 0)),
            pl.BlockSpec((1, SP, 16), lambda b, s: (b, s, 0)),
        ],
        out_shape=[
            jax.ShapeDtypeStruct((2, P, 8), jnp.float32),
            jax.ShapeDtypeStruct((2, P, 16), jnp.float32),
        ],
    )(xp, xp, xp, w9, bconv, whead, bhead, aconst, im_info)
    return probs, boxes


def _pack_weights(w_conv, b_conv, w_cls, b_cls, w_box, b_box):
    w9 = jnp.transpose(w_conv, (2, 3, 1, 0)).reshape(9, 256, 256)
    bconv = b_conv.reshape(1, 256)
    whead = jnp.zeros((256, 48), jnp.float32)
    bhead = jnp.zeros((1, 48), jnp.float32)
    wc = w_cls[:, :, 0, 0]
    wb = w_box[:, :, 0, 0]
    for a in range(_A):
        whead = whead.at[:, 0 + a].set(wc[2 * a])
        whead = whead.at[:, 8 + a].set(wc[2 * a + 1])
        bhead = bhead.at[0, 0 + a].set(b_cls[2 * a])
        bhead = bhead.at[0, 8 + a].set(b_cls[2 * a + 1])
        for c in range(4):
            whead = whead.at[:, 16 + 8 * c + a].set(wb[4 * a + c])
            bhead = bhead.at[0, 16 + 8 * c + a].set(b_box[4 * a + c])
    return w9, bconv, whead, bhead


_NMS_PAD = 4096
_NMS_ROWS = _NMS_PAD // 128


def _nms_body(x1_ref, y1_ref, x2_ref, y2_ref, keep_ref, area_ref, pbb_ref):
    area_ref[...] = (jnp.maximum(x2_ref[0] - x1_ref[0], 0.0) *
                     jnp.maximum(y2_ref[0] - y1_ref[0], 0.0))
    keep_ref[0] = jnp.ones((_NMS_ROWS, 128), jnp.float32)

    si = jax.lax.broadcasted_iota(jnp.int32, (128, 128), 0)
    li = jax.lax.broadcasted_iota(jnp.int32, (128, 128), 1)
    ident = (si == li).astype(jnp.float32)
    lane_gt_sub = (li > si).astype(jnp.float32)
    lane1 = jax.lax.broadcasted_iota(jnp.int32, (1, 128), 1)

    def col(row):
        return jax.lax.dot_general(
            ident, row, (((1,), (1,)), ((), ())),
            preferred_element_type=jnp.float32)

    def iou_mat(c_x1, c_y1, c_x2, c_y2, c_ar, r_x1, r_y1, r_x2, r_y2, r_ar):
        xx1 = jnp.maximum(c_x1, r_x1)
        yy1 = jnp.maximum(c_y1, r_y1)
        xx2 = jnp.minimum(c_x2, r_x2)
        yy2 = jnp.minimum(c_y2, r_y2)
        inter = jnp.maximum(xx2 - xx1, 0.0) * jnp.maximum(yy2 - yy1, 0.0)
        return inter / jnp.maximum(c_ar + r_ar - inter, 1e-9)

    def outer(b, _):
        r_x1 = x1_ref[0, pl.ds(b, 1), :]
        r_y1 = y1_ref[0, pl.ds(b, 1), :]
        r_x2 = x2_ref[0, pl.ds(b, 1), :]
        r_y2 = y2_ref[0, pl.ds(b, 1), :]
        r_ar = area_ref[pl.ds(b, 1), :]
        c_x1 = col(r_x1)
        c_y1 = col(r_y1)
        c_x2 = col(r_x2)
        c_y2 = col(r_y2)
        c_ar = col(r_ar)
        iou_bb = iou_mat(c_x1, c_y1, c_x2, c_y2, c_ar,
                         r_x1, r_y1, r_x2, r_y2, r_ar)
        pbb_ref[...] = (iou_bb > _NMS_THRESH).astype(jnp.float32) * lane_gt_sub

        def intra(i, kb):
            onehot = (lane1 == i).astype(jnp.float32)
            ki = jnp.sum(kb * onehot)
            row_i = pbb_ref[pl.ds(i, 1), :]
            return kb * (1.0 - row_i * ki)

        keep_b = lax.fori_loop(0, 128, intra, keep_ref[0, pl.ds(b, 1), :])
        keep_ref[0, pl.ds(b, 1), :] = keep_b

        def cross(c, _2):
            s_x1 = x1_ref[0, pl.ds(c, 1), :]
            s_y1 = y1_ref[0, pl.ds(c, 1), :]
            s_x2 = x2_ref[0, pl.ds(c, 1), :]
            s_y2 = y2_ref[0, pl.ds(c, 1), :]
            s_ar = area_ref[pl.ds(c, 1), :]
            iou_bc = iou_mat(c_x1, c_y1, c_x2, c_y2, c_ar,
                             s_x1, s_y1, s_x2, s_y2, s_ar)
            p_bc = (iou_bc > _NMS_THRESH).astype(jnp.float32)
            cnt = jax.lax.dot_general(
                keep_b, p_bc, (((1,), (0,)), ((), ())),
                preferred_element_type=jnp.float32)
            kc = keep_ref[0, pl.ds(c, 1), :] * (cnt <= 0.5).astype(jnp.float32)
            keep_ref[0, pl.ds(c, 1), :] = kc
            return 0

        lax.fori_loop(b + 1, _NMS_ROWS, cross, 0)
        return 0

    lax.fori_loop(0, _NMS_ROWS, outer, 0)


def _nms_keep_pallas(sboxes):
    planes = [sboxes[:, :, c].reshape(2, _NMS_ROWS, 128) for c in range(4)]
    keep = pl.pallas_call(
        _nms_body,
        grid=(2,),
        in_specs=[pl.BlockSpec((1, _NMS_ROWS, 128), lambda b: (b, 0, 0))] * 4,
        out_specs=pl.BlockSpec((1, _NMS_ROWS, 128), lambda b: (b, 0, 0)),
        out_shape=jax.ShapeDtypeStruct((2, _NMS_ROWS, 128), jnp.float32),
        scratch_shapes=[pltpu.VMEM((_NMS_ROWS, 128), jnp.float32),
                        pltpu.VMEM((128, 128), jnp.float32)],
    )(*planes)
    return keep.reshape(2, _NMS_PAD)


def kernel(feat0, feat1, feat2, feat3, feat4, im_info,
           w_conv, b_conv, w_cls, b_cls, w_box, b_box):
    feats = [feat0, feat1, feat2, feat3, feat4]
    w9, bconv, whead, bhead = _pack_weights(
        w_conv, b_conv, w_cls, b_cls, w_box, b_box)

    num_levels = len(feats)
    probs_l, boxes_l = [], []
    off = 2 ** (num_levels - 1)
    for x in feats:
        H, W = x.shape[2], x.shape[3]
        stride = 4 * off
        off //= 2
        p, bx = _head_level(x, w9, bconv, whead, bhead, im_info, H, W, stride)
        probs_l.append(p)
        boxes_l.append(bx)

    boxes_all, scores_all = [], []
    for l, x in enumerate(feats):
        H, W = x.shape[2], x.shape[3]
        P = H * W
        prob = probs_l[l][:, :, 0:3].reshape(2, P * _A)
        k = min(_PRE_NMS, P * _A)
        top_s, top_i = lax.top_k(prob, k)
        p_i = top_i // _A
        a_i = top_i % _A
        bflat = boxes_l[l].reshape(2, P * 16)
        base = p_i * 16 + a_i
        dec = jnp.stack([
            jnp.take_along_axis(bflat, base, axis=1),
            jnp.take_along_axis(bflat, base + 4, axis=1),
            jnp.take_along_axis(bflat, base + 8, axis=1),
            jnp.take_along_axis(bflat, base + 12, axis=1)], axis=2)
        boxes_all.append(dec)
        scores_all.append(top_s)
    boxes_cat = jnp.concatenate(boxes_all, axis=1)
    scores_cat = jnp.concatenate(scores_all, axis=1)
    ncand = scores_cat.shape[1]

    order = jnp.argsort(-scores_cat, axis=1)
    sboxes = jnp.take_along_axis(boxes_cat, order[:, :, None], axis=1)
    sscores = jnp.take_along_axis(scores_cat, order, axis=1)
    sboxes_p = jnp.pad(sboxes, ((0, 0), (0, _NMS_PAD - ncand), (0, 0)))

    keep = _nms_keep_pallas(sboxes_p)[:, :ncand] > 0.5
    masked = jnp.where(keep, sscores, -1.0)
    _, sel = lax.top_k(masked, _POST_NMS)
    final = jnp.take_along_axis(sboxes, sel[:, :, None], axis=1)
    bcol = jnp.tile(jnp.array([[0.0], [1.0]], jnp.float32)[:, None, :],
                    (1, _POST_NMS, 1))
    return jnp.concatenate([bcol, final], axis=2).reshape(2 * _POST_NMS, 5)
